# Initial kernel scaffold; baseline (speedup 1.0000x reference)
import jax
import jax.numpy as jnp
from jax.experimental import pallas as pl


def kernel(x, w_gate, b_gate, expert_biases):
    raise NotImplementedError("write your pallas kernel here")


# speedup vs baseline: 3.0350x; 3.0350x over previous
import functools

import jax
import jax.numpy as jnp
from jax.experimental import pallas as pl

T_BLK = 512
E = 64
K = 8


def _router_kernel(x_ref, w_ref, bias_ref, out_ref):
    logits = jnp.dot(x_ref[...], w_ref[...], preferred_element_type=jnp.float32)
    logits = logits + bias_ref[0:1, :]

    lane = jax.lax.broadcasted_iota(jnp.int32, logits.shape, 1)
    rank = jnp.zeros(logits.shape, dtype=jnp.int32)
    for j in range(E):
        lj = logits[:, j : j + 1]
        gt = (lj > logits).astype(jnp.int32)
        eq_before = jnp.logical_and(lj == logits, lane > j).astype(jnp.int32)
        rank = rank + gt + eq_before
    selected = rank < K

    m = jnp.max(logits, axis=1, keepdims=True)
    ex = jnp.where(selected, jnp.exp(logits - m), 0.0)
    z = jnp.sum(ex, axis=1, keepdims=True)
    out_ref[...] = ex / z


@jax.jit
def kernel(x, w_gate, b_gate, expert_biases):
    T, D = x.shape
    bias = jnp.broadcast_to((b_gate + expert_biases)[None, :], (8, E))
    grid = (T // T_BLK,)
    return pl.pallas_call(
        _router_kernel,
        grid=grid,
        in_specs=[
            pl.BlockSpec((T_BLK, D), lambda i: (i, 0)),
            pl.BlockSpec((D, E), lambda i: (0, 0)),
            pl.BlockSpec((8, E), lambda i: (0, 0)),
        ],
        out_specs=pl.BlockSpec((T_BLK, E), lambda i: (i, 0)),
        out_shape=jax.ShapeDtypeStruct((T, E), x.dtype),
    )(x, w_gate, bias)


# speedup vs baseline: 5.7440x; 1.8926x over previous
import functools

import jax
import jax.numpy as jnp
from jax.experimental import pallas as pl

T_BLK = 512
E = 64
K = 8


def _router_kernel(x_ref, w_ref, bias_ref, out_ref):
    logits = jnp.dot(x_ref[...], w_ref[...], preferred_element_type=jnp.float32)
    logits = logits + bias_ref[0:1, :]

    lane = jax.lax.broadcasted_iota(jnp.int32, logits.shape, 1).astype(jnp.float32)
    cur = logits
    selected = jnp.zeros(logits.shape, dtype=jnp.bool_)
    m0 = None
    for _ in range(K):
        m = jnp.max(cur, axis=1, keepdims=True)
        if m0 is None:
            m0 = m
        idx = jnp.min(jnp.where(cur == m, lane, float(E)), axis=1, keepdims=True)
        pick = lane == idx
        selected = jnp.logical_or(selected, pick)
        cur = jnp.where(pick, -jnp.inf, cur)

    ex = jnp.where(selected, jnp.exp(logits - m0), 0.0)
    z = jnp.sum(ex, axis=1, keepdims=True)
    out_ref[...] = ex / z


@jax.jit
def kernel(x, w_gate, b_gate, expert_biases):
    T, D = x.shape
    bias = jnp.broadcast_to((b_gate + expert_biases)[None, :], (8, E))
    grid = (T // T_BLK,)
    return pl.pallas_call(
        _router_kernel,
        grid=grid,
        in_specs=[
            pl.BlockSpec((T_BLK, D), lambda i: (i, 0)),
            pl.BlockSpec((D, E), lambda i: (0, 0)),
            pl.BlockSpec((8, E), lambda i: (0, 0)),
        ],
        out_specs=pl.BlockSpec((T_BLK, E), lambda i: (i, 0)),
        out_shape=jax.ShapeDtypeStruct((T, E), x.dtype),
    )(x, w_gate, bias)


# speedup vs baseline: 6.2615x; 1.0901x over previous
import functools

import jax
import jax.numpy as jnp
from jax.experimental import pallas as pl

T_BLK = 512
E = 64
K = 8


def _router_kernel(x_ref, w_ref, bias_ref, out_ref):
    logits = jnp.dot(x_ref[...], w_ref[...], preferred_element_type=jnp.float32)
    logits = logits + bias_ref[0:1, :]

    cur = logits
    m0 = None
    for _ in range(K):
        m = jnp.max(cur, axis=1, keepdims=True)
        if m0 is None:
            m0 = m
        cur = jnp.where(cur == m, -jnp.inf, cur)

    ex = jnp.where(cur < logits, jnp.exp(logits - m0), 0.0)
    z = jnp.sum(ex, axis=1, keepdims=True)
    out_ref[...] = ex / z


@jax.jit
def kernel(x, w_gate, b_gate, expert_biases):
    T, D = x.shape
    bias = jnp.broadcast_to((b_gate + expert_biases)[None, :], (8, E))
    grid = (T // T_BLK,)
    return pl.pallas_call(
        _router_kernel,
        grid=grid,
        in_specs=[
            pl.BlockSpec((T_BLK, D), lambda i: (i, 0)),
            pl.BlockSpec((D, E), lambda i: (0, 0)),
            pl.BlockSpec((8, E), lambda i: (0, 0)),
        ],
        out_specs=pl.BlockSpec((T_BLK, E), lambda i: (i, 0)),
        out_shape=jax.ShapeDtypeStruct((T, E), x.dtype),
    )(x, w_gate, bias)


# speedup vs baseline: 6.7181x; 1.0729x over previous
import functools

import jax
import jax.numpy as jnp
from jax.experimental import pallas as pl

T_BLK = 1024
E = 64
K = 8


def _router_kernel(x_ref, w_ref, bias_ref, out_ref):
    logits = jnp.dot(x_ref[...], w_ref[...], preferred_element_type=jnp.float32)
    logits = logits + bias_ref[0:1, :]

    cur = logits
    m0 = None
    for _ in range(K):
        m = jnp.max(cur, axis=1, keepdims=True)
        if m0 is None:
            m0 = m
        cur = jnp.where(cur == m, -jnp.inf, cur)

    ex = jnp.where(cur < logits, jnp.exp(logits - m0), 0.0)
    z = jnp.sum(ex, axis=1, keepdims=True)
    out_ref[...] = ex / z


@jax.jit
def kernel(x, w_gate, b_gate, expert_biases):
    T, D = x.shape
    bias = jnp.broadcast_to((b_gate + expert_biases)[None, :], (8, E))
    grid = (T // T_BLK,)
    return pl.pallas_call(
        _router_kernel,
        grid=grid,
        in_specs=[
            pl.BlockSpec((T_BLK, D), lambda i: (i, 0)),
            pl.BlockSpec((D, E), lambda i: (0, 0)),
            pl.BlockSpec((8, E), lambda i: (0, 0)),
        ],
        out_specs=pl.BlockSpec((T_BLK, E), lambda i: (i, 0)),
        out_shape=jax.ShapeDtypeStruct((T, E), x.dtype),
    )(x, w_gate, bias)
